# Initial kernel scaffold; baseline (speedup 1.0000x reference)
import jax
import jax.numpy as jnp
from jax.experimental import pallas as pl


def kernel(x, edge_index, W1l, b1l, W1r, W2l, b2l, W2r):
    raise NotImplementedError("write your pallas kernel here")


# speedup vs baseline: 9.5275x; 9.5275x over previous
import functools

import jax
import jax.numpy as jnp
from jax import lax
from jax.experimental import pallas as pl
from jax.experimental.pallas import tpu as pltpu
from jax.experimental.pallas import tpu_sc as plsc

N = 10000
NPAD = 10240
E = 320000
D_IN = 128
DH = 16
CHUNK = 128
NCHUNK = E // CHUNK
NWORK = 32
MAXITER = -(-NCHUNK // NWORK)
ROWS_PER_SUB = NPAD // 16


def _sc_edge_pass_body(with_deg, table, src, dst, *refs):
    if with_deg:
        (acc0, acc1, deg0, deg1, idx_v, rows_v, ones_v, zrow_v, zdeg_v,
         acc_sh, deg_sh, sem) = refs
    else:
        (acc0, acc1, idx_v, rows_v, zrow_v, acc_sh, sem) = refs

    cid = lax.axis_index("c")
    sid = lax.axis_index("s")
    wid = sid * 2 + cid

    zf16 = jnp.zeros((16,), jnp.float32)

    def _zero_zrow(i, _):
        zrow_v[i, :] = zf16
        return 0
    lax.fori_loop(0, 128, _zero_zrow, 0)

    if with_deg:
        def _zero_zdeg(i, _):
            zdeg_v[pl.ds(i * 16, 16)] = zf16
            return 0
        lax.fori_loop(0, ROWS_PER_SUB // 16, _zero_zdeg, 0)

        of16 = jnp.ones((16,), jnp.float32)

        def _fill_ones(i, _):
            ones_v[pl.ds(i * 16, 16)] = of16
            return 0
        lax.fori_loop(0, CHUNK // 16, _fill_ones, 0)

    rbase = sid * ROWS_PER_SUB
    for j in range(ROWS_PER_SUB // 128):
        pltpu.sync_copy(zrow_v, acc_sh.at[pl.ds(rbase + j * 128, 128)])
    if with_deg:
        pltpu.sync_copy(zdeg_v, deg_sh.at[pl.ds(rbase, ROWS_PER_SUB)])
    plsc.subcore_barrier()

    def _chunk(i, _):
        c = wid + i * NWORK

        @pl.when(c < NCHUNK)
        def _():
            base = c * CHUNK
            pltpu.sync_copy(src.at[pl.ds(base, CHUNK)], idx_v.at[0])
            pltpu.sync_copy(dst.at[pl.ds(base, CHUNK)], idx_v.at[1])
            pltpu.async_copy(table.at[idx_v.at[0]], rows_v, sem).wait()
            pltpu.sync_copy(rows_v, acc_sh.at[idx_v.at[1]], add=True)
            if with_deg:
                pltpu.sync_copy(ones_v, deg_sh.at[idx_v.at[1]], add=True)
        return 0

    lax.fori_loop(0, MAXITER, _chunk, 0)
    plsc.subcore_barrier()

    acc_out = [acc0, acc1]
    for core in range(2):
        @pl.when(cid == core)
        def _():
            pltpu.sync_copy(acc_sh.at[pl.ds(rbase, ROWS_PER_SUB)],
                            acc_out[core].at[pl.ds(rbase, ROWS_PER_SUB)])
            if with_deg:
                deg_out = [deg0, deg1][core]
                pltpu.sync_copy(deg_sh.at[pl.ds(rbase, ROWS_PER_SUB)],
                                deg_out.at[pl.ds(rbase, ROWS_PER_SUB)])


def _make_sc_pass(with_deg):
    out_type = [jax.ShapeDtypeStruct((NPAD, DH), jnp.float32),
                jax.ShapeDtypeStruct((NPAD, DH), jnp.float32)]
    scratch = [pltpu.VMEM((2, CHUNK), jnp.int32),
               pltpu.VMEM((CHUNK, DH), jnp.float32)]
    if with_deg:
        out_type += [jax.ShapeDtypeStruct((NPAD,), jnp.float32),
                     jax.ShapeDtypeStruct((NPAD,), jnp.float32)]
        scratch += [pltpu.VMEM((CHUNK,), jnp.float32)]
    scratch += [pltpu.VMEM((128, DH), jnp.float32)]
    if with_deg:
        scratch += [pltpu.VMEM((ROWS_PER_SUB,), jnp.float32)]
    scratch += [pltpu.VMEM_SHARED((NPAD, DH), jnp.float32)]
    if with_deg:
        scratch += [pltpu.VMEM_SHARED((NPAD,), jnp.float32)]
    scratch += [pltpu.SemaphoreType.DMA]

    mesh = plsc.VectorSubcoreMesh(core_axis_name="c", subcore_axis_name="s")
    return pl.kernel(
        functools.partial(_sc_edge_pass_body, with_deg),
        out_type=out_type,
        mesh=mesh,
        scratch_types=scratch,
        compiler_params=pltpu.CompilerParams(use_tc_tiling_on_sc=False),
        name=f"sc_edge_pass_deg{int(with_deg)}",
    )


_sc_pass_deg = _make_sc_pass(True)
_sc_pass = _make_sc_pass(False)


def _tc1_body(x_ref, w1l_ref, w1r_ref, y1_ref, z1_ref):
    x = x_ref[...]
    y1_ref[...] = jnp.dot(x, w1l_ref[...], preferred_element_type=jnp.float32)
    z1_ref[...] = jnp.dot(x, w1r_ref[...], preferred_element_type=jnp.float32)


def _tc2_body(acc0_ref, acc1_ref, deg0_ref, deg1_ref, z1_ref, b1l_ref,
              w2l_ref, w2r_ref, y2_ref, z2_ref):
    acc = acc0_ref[...] + acc1_ref[...]
    deg = jnp.maximum(deg0_ref[...] + deg1_ref[...], 1.0)
    agg = acc[:N] / deg[:N, None]
    h = jnp.maximum(agg + b1l_ref[...] + z1_ref[...], 0.0)
    y2_ref[...] = jnp.dot(h, w2l_ref[...], preferred_element_type=jnp.float32)
    z2_ref[...] = jnp.dot(h, w2r_ref[...], preferred_element_type=jnp.float32)


def _tc3_body(acc0_ref, acc1_ref, deg0_ref, deg1_ref, z2_ref, b2l_ref,
              out_ref):
    acc = acc0_ref[...] + acc1_ref[...]
    deg = jnp.maximum(deg0_ref[...] + deg1_ref[...], 1.0)
    out_ref[...] = acc[:N] / deg[:N, None] + b2l_ref[...] + z2_ref[...]


_tc1 = pl.pallas_call(
    _tc1_body,
    out_shape=[jax.ShapeDtypeStruct((N, DH), jnp.float32),
               jax.ShapeDtypeStruct((N, DH), jnp.float32)],
)

_tc2 = pl.pallas_call(
    _tc2_body,
    out_shape=[jax.ShapeDtypeStruct((N, DH), jnp.float32),
               jax.ShapeDtypeStruct((N, DH), jnp.float32)],
)

_tc3 = pl.pallas_call(
    _tc3_body,
    out_shape=jax.ShapeDtypeStruct((N, DH), jnp.float32),
)


def kernel(x, edge_index, W1l, b1l, W1r, W2l, b2l, W2r):
    src = edge_index[0]
    dst = edge_index[1]

    W2l_p = jnp.zeros((DH, DH), jnp.float32).at[:, : W2l.shape[1]].set(W2l)
    W2r_p = jnp.zeros((DH, DH), jnp.float32).at[:, : W2r.shape[1]].set(W2r)
    b1l_p = b1l.reshape(1, DH)
    b2l_p = jnp.zeros((1, DH), jnp.float32).at[0, : b2l.shape[0]].set(b2l)

    y1, z1 = _tc1(x, W1l, W1r)
    a0, a1, d0, d1 = _sc_pass_deg(y1, src, dst)
    y2, z2 = _tc2(a0, a1, d0, d1, z1, b1l_p, W2l_p, W2r_p)
    b0, b1 = _sc_pass(y2, src, dst)
    out16 = _tc3(b0, b1, d0, d1, z2, b2l_p)
    return out16[:, : b2l.shape[0]]


# speedup vs baseline: 20.6582x; 2.1683x over previous
import functools

import jax
import jax.numpy as jnp
from jax import lax
from jax.experimental import pallas as pl
from jax.experimental.pallas import tpu as pltpu
from jax.experimental.pallas import tpu_sc as plsc

N = 10000
NPAD = 10240
E = 320000
D_IN = 128
DH = 16
CHUNK = 128
NCHUNK = E // CHUNK
NWORK = 32
NCH_BASE = NCHUNK // NWORK
NREM = NCHUNK - NCH_BASE * NWORK
ROWS_PER_SUB = NPAD // 16


def _sc_edge_pass_body(with_deg, table, src2d, dst2d, *refs):
    if with_deg:
        (acc0, acc1, deg0, deg1, sbuf, dbuf, rows0, rows1, ones_v, zrow_v,
         zdeg_v, acc_sh, deg_sh, semg0, semg1, sems0, sems1, semd0,
         semd1) = refs
    else:
        (acc0, acc1, sbuf, dbuf, rows0, rows1, zrow_v, acc_sh,
         semg0, semg1, sems0, sems1) = refs

    cid = lax.axis_index("c")
    sid = lax.axis_index("s")
    wid = sid * 2 + cid

    rows = [rows0, rows1]
    semg = [semg0, semg1]
    sems = [sems0, sems1]
    semd = [semd0, semd1] if with_deg else None

    zf16 = jnp.zeros((16,), jnp.float32)

    def _zero_zrow(i, _):
        zrow_v[i, :] = zf16
        return 0
    lax.fori_loop(0, 128, _zero_zrow, 0)

    if with_deg:
        def _zero_zdeg(i, _):
            zdeg_v[pl.ds(i * 16, 16)] = zf16
            return 0
        lax.fori_loop(0, ROWS_PER_SUB // 16, _zero_zdeg, 0)

        of16 = jnp.ones((16,), jnp.float32)

        def _fill_ones(i, _):
            ones_v[pl.ds(i * 16, 16)] = of16
            return 0
        lax.fori_loop(0, CHUNK // 16, _fill_ones, 0)

    rbase = sid * ROWS_PER_SUB
    for j in range(ROWS_PER_SUB // 128):
        pltpu.sync_copy(zrow_v, acc_sh.at[pl.ds(rbase + j * 128, 128)])
    if with_deg:
        pltpu.sync_copy(zdeg_v, deg_sh.at[pl.ds(rbase, ROWS_PER_SUB)])

    cstart = wid * NCH_BASE
    pltpu.sync_copy(src2d.at[pl.ds(cstart, NCH_BASE)],
                    sbuf.at[pl.ds(0, NCH_BASE)])
    pltpu.sync_copy(dst2d.at[pl.ds(cstart, NCH_BASE)],
                    dbuf.at[pl.ds(0, NCH_BASE)])

    @pl.when(wid < NREM)
    def _():
        extra = NWORK * NCH_BASE + wid
        pltpu.sync_copy(src2d.at[pl.ds(extra, 1)], sbuf.at[pl.ds(NCH_BASE, 1)])
        pltpu.sync_copy(dst2d.at[pl.ds(extra, 1)], dbuf.at[pl.ds(NCH_BASE, 1)])

    plsc.subcore_barrier()

    def _start_gather(i, b):
        return pltpu.async_copy(table.at[sbuf.at[i]], rows[b], semg[b])

    def _wait_gather(b):
        pltpu.make_async_copy(table.at[sbuf.at[0]], rows[b], semg[b]).wait()

    def _start_scatter(i, b):
        pltpu.async_copy(rows[b], acc_sh.at[dbuf.at[i]], sems[b], add=True)
        if with_deg:
            pltpu.async_copy(ones_v, deg_sh.at[dbuf.at[i]], semd[b], add=True)

    def _wait_scatter(b):
        pltpu.make_async_copy(rows[b], acc_sh.at[dbuf.at[0]], sems[b]).wait()
        if with_deg:
            pltpu.make_async_copy(ones_v, deg_sh.at[dbuf.at[0]],
                                  semd[b]).wait()

    _start_gather(0, 0)

    @pl.loop(0, NCH_BASE // 2)
    def _pipe(j):
        i0 = 2 * j
        i1 = i0 + 1

        @pl.when(j > 0)
        def _():
            _wait_scatter(1)
        _start_gather(i1, 1)
        _wait_gather(0)
        _start_scatter(i0, 0)

        _wait_scatter(0)

        @pl.when(j < NCH_BASE // 2 - 1)
        def _():
            _start_gather(i0 + 2, 0)
        _wait_gather(1)
        _start_scatter(i1, 1)

    _wait_scatter(1)

    @pl.when(wid < NREM)
    def _():
        _start_gather(NCH_BASE, 0).wait()
        _start_scatter(NCH_BASE, 0)
        _wait_scatter(0)

    plsc.subcore_barrier()

    acc_out = [acc0, acc1]
    for core in range(2):
        @pl.when(cid == core)
        def _():
            pltpu.sync_copy(acc_sh.at[pl.ds(rbase, ROWS_PER_SUB)],
                            acc_out[core].at[pl.ds(rbase, ROWS_PER_SUB)])
            if with_deg:
                deg_out = [deg0, deg1][core]
                pltpu.sync_copy(deg_sh.at[pl.ds(rbase, ROWS_PER_SUB)],
                                deg_out.at[pl.ds(rbase, ROWS_PER_SUB)])


def _make_sc_pass(with_deg):
    out_type = [jax.ShapeDtypeStruct((NPAD, DH), jnp.float32),
                jax.ShapeDtypeStruct((NPAD, DH), jnp.float32)]
    if with_deg:
        out_type += [jax.ShapeDtypeStruct((NPAD,), jnp.float32),
                     jax.ShapeDtypeStruct((NPAD,), jnp.float32)]
    scratch = [pltpu.VMEM((NCH_BASE + 1, CHUNK), jnp.int32),
               pltpu.VMEM((NCH_BASE + 1, CHUNK), jnp.int32),
               pltpu.VMEM((CHUNK, DH), jnp.float32),
               pltpu.VMEM((CHUNK, DH), jnp.float32)]
    if with_deg:
        scratch += [pltpu.VMEM((CHUNK,), jnp.float32)]
    scratch += [pltpu.VMEM((128, DH), jnp.float32)]
    if with_deg:
        scratch += [pltpu.VMEM((ROWS_PER_SUB,), jnp.float32)]
    scratch += [pltpu.VMEM_SHARED((NPAD, DH), jnp.float32)]
    if with_deg:
        scratch += [pltpu.VMEM_SHARED((NPAD,), jnp.float32)]
    nsem = 6 if with_deg else 4
    scratch += [pltpu.SemaphoreType.DMA] * nsem

    mesh = plsc.VectorSubcoreMesh(core_axis_name="c", subcore_axis_name="s")
    return pl.kernel(
        functools.partial(_sc_edge_pass_body, with_deg),
        out_type=out_type,
        mesh=mesh,
        scratch_types=scratch,
        compiler_params=pltpu.CompilerParams(use_tc_tiling_on_sc=False),
        name=f"sc_edge_pass_deg{int(with_deg)}",
    )


_sc_pass_deg = _make_sc_pass(True)
_sc_pass = _make_sc_pass(False)


def _tc1_body(x_ref, w1l_ref, w1r_ref, y1_ref, z1_ref):
    x = x_ref[...]
    y1_ref[...] = jnp.dot(x, w1l_ref[...], preferred_element_type=jnp.float32)
    z1_ref[...] = jnp.dot(x, w1r_ref[...], preferred_element_type=jnp.float32)


def _tc2_body(acc0_ref, acc1_ref, deg0_ref, deg1_ref, z1_ref, b1l_ref,
              w2l_ref, w2r_ref, y2_ref, z2_ref):
    acc = acc0_ref[...] + acc1_ref[...]
    deg = jnp.maximum(deg0_ref[...] + deg1_ref[...], 1.0)
    agg = acc[:N] / deg[:N, None]
    h = jnp.maximum(agg + b1l_ref[...] + z1_ref[...], 0.0)
    y2_ref[...] = jnp.dot(h, w2l_ref[...], preferred_element_type=jnp.float32)
    z2_ref[...] = jnp.dot(h, w2r_ref[...], preferred_element_type=jnp.float32)


def _tc3_body(acc0_ref, acc1_ref, deg0_ref, deg1_ref, z2_ref, b2l_ref,
              out_ref):
    acc = acc0_ref[...] + acc1_ref[...]
    deg = jnp.maximum(deg0_ref[...] + deg1_ref[...], 1.0)
    out_ref[...] = acc[:N] / deg[:N, None] + b2l_ref[...] + z2_ref[...]


_tc1 = pl.pallas_call(
    _tc1_body,
    out_shape=[jax.ShapeDtypeStruct((N, DH), jnp.float32),
               jax.ShapeDtypeStruct((N, DH), jnp.float32)],
)

_tc2 = pl.pallas_call(
    _tc2_body,
    out_shape=[jax.ShapeDtypeStruct((N, DH), jnp.float32),
               jax.ShapeDtypeStruct((N, DH), jnp.float32)],
)

_tc3 = pl.pallas_call(
    _tc3_body,
    out_shape=jax.ShapeDtypeStruct((N, DH), jnp.float32),
)


def kernel(x, edge_index, W1l, b1l, W1r, W2l, b2l, W2r):
    src2d = edge_index[0].reshape(NCHUNK, CHUNK)
    dst2d = edge_index[1].reshape(NCHUNK, CHUNK)

    W2l_p = jnp.zeros((DH, DH), jnp.float32).at[:, : W2l.shape[1]].set(W2l)
    W2r_p = jnp.zeros((DH, DH), jnp.float32).at[:, : W2r.shape[1]].set(W2r)
    b1l_p = b1l.reshape(1, DH)
    b2l_p = jnp.zeros((1, DH), jnp.float32).at[0, : b2l.shape[0]].set(b2l)

    y1, z1 = _tc1(x, W1l, W1r)
    a0, a1, d0, d1 = _sc_pass_deg(y1, src2d, dst2d)
    y2, z2 = _tc2(a0, a1, d0, d1, z1, b1l_p, W2l_p, W2r_p)
    b0, b1 = _sc_pass(y2, src2d, dst2d)
    out16 = _tc3(b0, b1, d0, d1, z2, b2l_p)
    return out16[:, : b2l.shape[0]]


# speedup vs baseline: 22.8447x; 1.1058x over previous
import functools

import jax
import jax.numpy as jnp
from jax import lax
from jax.experimental import pallas as pl
from jax.experimental.pallas import tpu as pltpu
from jax.experimental.pallas import tpu_sc as plsc

N = 10000
NPAD = 10240
E = 320000
D_IN = 128
DH = 16
CHUNK = 128
NCHUNK = E // CHUNK
NWORK = 32
NCH_BASE = NCHUNK // NWORK
NREM = NCHUNK - NCH_BASE * NWORK
ROWS_PER_SUB = NPAD // 16


def _sc_edge_pass_body(with_deg, table, ei3, *refs):
    if with_deg:
        (acc0, acc1, deg0, deg1, sdbuf, rows0, rows1, ones_v, zrow_v,
         zdeg_v, acc_sh, deg_sh, semg0, semg1, sems0, sems1, semd0,
         semd1) = refs
    else:
        (acc0, acc1, sdbuf, rows0, rows1, zrow_v, acc_sh,
         semg0, semg1, sems0, sems1) = refs

    cid = lax.axis_index("c")
    sid = lax.axis_index("s")
    wid = sid * 2 + cid

    rows = [rows0, rows1]
    semg = [semg0, semg1]
    sems = [sems0, sems1]
    semd = [semd0, semd1] if with_deg else None

    zf16 = jnp.zeros((16,), jnp.float32)

    def _zero_zrow(i, _):
        zrow_v[i, :] = zf16
        return 0
    lax.fori_loop(0, 128, _zero_zrow, 0)

    if with_deg:
        def _zero_zdeg(i, _):
            zdeg_v[pl.ds(i * 16, 16)] = zf16
            return 0
        lax.fori_loop(0, ROWS_PER_SUB // 16, _zero_zdeg, 0)

        of16 = jnp.ones((16,), jnp.float32)

        def _fill_ones(i, _):
            ones_v[pl.ds(i * 16, 16)] = of16
            return 0
        lax.fori_loop(0, CHUNK // 16, _fill_ones, 0)

    rbase = sid * ROWS_PER_SUB
    for j in range(ROWS_PER_SUB // 128):
        pltpu.sync_copy(zrow_v, acc_sh.at[pl.ds(rbase + j * 128, 128)])
    if with_deg:
        pltpu.sync_copy(zdeg_v, deg_sh.at[pl.ds(rbase, ROWS_PER_SUB)])

    cstart = wid * NCH_BASE
    pltpu.sync_copy(ei3.at[pl.ds(cstart, NCH_BASE)],
                    sdbuf.at[pl.ds(0, NCH_BASE)])

    @pl.when(wid < NREM)
    def _():
        extra = NWORK * NCH_BASE + wid
        pltpu.sync_copy(ei3.at[pl.ds(extra, 1)], sdbuf.at[pl.ds(NCH_BASE, 1)])

    plsc.subcore_barrier()

    def _start_gather(i, b):
        return pltpu.async_copy(table.at[sdbuf.at[i, 0]], rows[b], semg[b])

    def _wait_gather(b):
        pltpu.make_async_copy(table.at[sdbuf.at[0, 0]], rows[b],
                              semg[b]).wait()

    def _start_scatter(i, b):
        pltpu.async_copy(rows[b], acc_sh.at[sdbuf.at[i, 1]], sems[b], add=True)
        if with_deg:
            pltpu.async_copy(ones_v, deg_sh.at[sdbuf.at[i, 1]], semd[b],
                             add=True)

    def _wait_scatter(b):
        pltpu.make_async_copy(rows[b], acc_sh.at[sdbuf.at[0, 1]],
                              sems[b]).wait()
        if with_deg:
            pltpu.make_async_copy(ones_v, deg_sh.at[sdbuf.at[0, 1]],
                                  semd[b]).wait()

    _start_gather(0, 0)

    @pl.loop(0, NCH_BASE // 2)
    def _pipe(j):
        i0 = 2 * j
        i1 = i0 + 1

        @pl.when(j > 0)
        def _():
            _wait_scatter(1)
        _start_gather(i1, 1)
        _wait_gather(0)
        _start_scatter(i0, 0)

        _wait_scatter(0)

        @pl.when(j < NCH_BASE // 2 - 1)
        def _():
            _start_gather(i0 + 2, 0)
        _wait_gather(1)
        _start_scatter(i1, 1)

    _wait_scatter(1)

    @pl.when(wid < NREM)
    def _():
        _start_gather(NCH_BASE, 0).wait()
        _start_scatter(NCH_BASE, 0)
        _wait_scatter(0)

    plsc.subcore_barrier()

    acc_out = [acc0, acc1]
    for core in range(2):
        @pl.when(cid == core)
        def _():
            pltpu.sync_copy(acc_sh.at[pl.ds(rbase, ROWS_PER_SUB)],
                            acc_out[core].at[pl.ds(rbase, ROWS_PER_SUB)])
            if with_deg:
                deg_out = [deg0, deg1][core]
                pltpu.sync_copy(deg_sh.at[pl.ds(rbase, ROWS_PER_SUB)],
                                deg_out.at[pl.ds(rbase, ROWS_PER_SUB)])


def _make_sc_pass(with_deg):
    out_type = [jax.ShapeDtypeStruct((NPAD, DH), jnp.float32),
                jax.ShapeDtypeStruct((NPAD, DH), jnp.float32)]
    if with_deg:
        out_type += [jax.ShapeDtypeStruct((NPAD,), jnp.float32),
                     jax.ShapeDtypeStruct((NPAD,), jnp.float32)]
    scratch = [pltpu.VMEM((NCH_BASE + 1, 2, CHUNK), jnp.int32),
               pltpu.VMEM((CHUNK, DH), jnp.float32),
               pltpu.VMEM((CHUNK, DH), jnp.float32)]
    if with_deg:
        scratch += [pltpu.VMEM((CHUNK,), jnp.float32)]
    scratch += [pltpu.VMEM((128, DH), jnp.float32)]
    if with_deg:
        scratch += [pltpu.VMEM((ROWS_PER_SUB,), jnp.float32)]
    scratch += [pltpu.VMEM_SHARED((NPAD, DH), jnp.float32)]
    if with_deg:
        scratch += [pltpu.VMEM_SHARED((NPAD,), jnp.float32)]
    nsem = 6 if with_deg else 4
    scratch += [pltpu.SemaphoreType.DMA] * nsem

    mesh = plsc.VectorSubcoreMesh(core_axis_name="c", subcore_axis_name="s")
    return pl.kernel(
        functools.partial(_sc_edge_pass_body, with_deg),
        out_type=out_type,
        mesh=mesh,
        scratch_types=scratch,
        compiler_params=pltpu.CompilerParams(use_tc_tiling_on_sc=False),
        name=f"sc_edge_pass_deg{int(with_deg)}",
    )


_sc_pass_deg = _make_sc_pass(True)
_sc_pass = _make_sc_pass(False)


def _tc1_body(x_ref, w1l_ref, w1r_ref, y1_ref, z1_ref):
    x = x_ref[...]
    y1_ref[...] = jnp.dot(x, w1l_ref[...], preferred_element_type=jnp.float32)
    z1_ref[...] = jnp.dot(x, w1r_ref[...], preferred_element_type=jnp.float32)


def _tc2_body(acc0_ref, acc1_ref, deg0_ref, deg1_ref, z1_ref, b1l_ref,
              w2l_ref, w2r_ref, y2_ref, z2_ref):
    acc = acc0_ref[...] + acc1_ref[...]
    deg = jnp.maximum(deg0_ref[...] + deg1_ref[...], 1.0)
    agg = acc[:N] / deg[:N, None]
    h = jnp.maximum(agg + b1l_ref[...] + z1_ref[...], 0.0)
    y2_ref[...] = jnp.dot(h, w2l_ref[...], preferred_element_type=jnp.float32)
    z2_ref[...] = jnp.dot(h, w2r_ref[...], preferred_element_type=jnp.float32)


def _tc3_body(acc0_ref, acc1_ref, deg0_ref, deg1_ref, z2_ref, b2l_ref,
              out_ref):
    acc = acc0_ref[...] + acc1_ref[...]
    deg = jnp.maximum(deg0_ref[...] + deg1_ref[...], 1.0)
    full = acc[:N] / deg[:N, None] + z2_ref[...]
    out_ref[...] = full[:, : out_ref.shape[1]] + b2l_ref[...]


_tc1 = pl.pallas_call(
    _tc1_body,
    out_shape=[jax.ShapeDtypeStruct((N, DH), jnp.float32),
               jax.ShapeDtypeStruct((N, DH), jnp.float32)],
)

_tc2 = pl.pallas_call(
    _tc2_body,
    out_shape=[jax.ShapeDtypeStruct((N, DH), jnp.float32),
               jax.ShapeDtypeStruct((N, DH), jnp.float32)],
)

_tc3 = pl.pallas_call(
    _tc3_body,
    out_shape=jax.ShapeDtypeStruct((N, 7), jnp.float32),
)


def kernel(x, edge_index, W1l, b1l, W1r, W2l, b2l, W2r):
    ei3 = edge_index.reshape(2, NCHUNK, CHUNK).transpose(1, 0, 2)

    W2l_p = jnp.pad(W2l, ((0, 0), (0, DH - W2l.shape[1])))
    W2r_p = jnp.pad(W2r, ((0, 0), (0, DH - W2r.shape[1])))
    b1l_p = b1l.reshape(1, DH)
    b2l_p = b2l.reshape(1, 7)

    y1, z1 = _tc1(x, W1l, W1r)
    a0, a1, d0, d1 = _sc_pass_deg(y1, ei3)
    y2, z2 = _tc2(a0, a1, d0, d1, z1, b1l_p, W2l_p, W2r_p)
    b0, b1 = _sc_pass(y2, ei3)
    return _tc3(b0, b1, d0, d1, z2, b2l_p)


# speedup vs baseline: 28.1332x; 1.2315x over previous
import functools

import jax
import jax.numpy as jnp
from jax import lax
from jax.experimental import pallas as pl
from jax.experimental.pallas import tpu as pltpu
from jax.experimental.pallas import tpu_sc as plsc

N = 10000
NPAD = 10240
E = 320000
D_IN = 128
DH = 16
CHUNK = 128
NCHUNK = E // CHUNK
NWORK = 32
NCH_BASE = NCHUNK // NWORK
NREM = NCHUNK - NCH_BASE * NWORK
ROWS_PER_SUB = NPAD // 16


def _sc_edge_pass_body(with_deg, table, ei3, *refs):
    if with_deg:
        (acc0, acc1, deg0, deg1, sdbuf, rows0, rows1, ones_v, zrow_v,
         zdeg_v, dv, dv16, acc_sh, deg_sh, semg0, semg1, sems0, sems1, semd0,
         semd1) = refs
    else:
        (acc0, acc1, sdbuf, rows0, rows1, zrow_v, acc_sh,
         semg0, semg1, sems0, sems1) = refs

    cid = lax.axis_index("c")
    sid = lax.axis_index("s")
    wid = sid * 2 + cid

    rows = [rows0, rows1]
    semg = [semg0, semg1]
    sems = [sems0, sems1]
    semd = [semd0, semd1] if with_deg else None

    zf16 = jnp.zeros((16,), jnp.float32)

    def _zero_zrow(i, _):
        zrow_v[i, :] = zf16
        return 0
    lax.fori_loop(0, 128, _zero_zrow, 0)

    if with_deg:
        def _zero_zdeg(i, _):
            zdeg_v[pl.ds(i * 16, 16)] = zf16
            return 0
        lax.fori_loop(0, ROWS_PER_SUB // 16, _zero_zdeg, 0)

        of16 = jnp.ones((16,), jnp.float32)

        def _fill_ones(i, _):
            ones_v[pl.ds(i * 16, 16)] = of16
            return 0
        lax.fori_loop(0, CHUNK // 16, _fill_ones, 0)

    rbase = sid * ROWS_PER_SUB
    for j in range(ROWS_PER_SUB // 128):
        pltpu.sync_copy(zrow_v, acc_sh.at[pl.ds(rbase + j * 128, 128)])
    if with_deg:
        pltpu.sync_copy(zdeg_v, deg_sh.at[pl.ds(rbase, ROWS_PER_SUB)])

    cstart = wid * NCH_BASE
    pltpu.sync_copy(ei3.at[pl.ds(cstart, NCH_BASE)],
                    sdbuf.at[pl.ds(0, NCH_BASE)])

    @pl.when(wid < NREM)
    def _():
        extra = NWORK * NCH_BASE + wid
        pltpu.sync_copy(ei3.at[pl.ds(extra, 1)], sdbuf.at[pl.ds(NCH_BASE, 1)])

    plsc.subcore_barrier()

    def _start_gather(i, b):
        return pltpu.async_copy(table.at[sdbuf.at[i, 0]], rows[b], semg[b])

    def _wait_gather(b):
        pltpu.make_async_copy(table.at[sdbuf.at[0, 0]], rows[b],
                              semg[b]).wait()

    def _start_scatter(i, b):
        pltpu.async_copy(rows[b], acc_sh.at[sdbuf.at[i, 1]], sems[b], add=True)
        if with_deg:
            pltpu.async_copy(ones_v, deg_sh.at[sdbuf.at[i, 1]], semd[b],
                             add=True)

    def _wait_scatter(b):
        pltpu.make_async_copy(rows[b], acc_sh.at[sdbuf.at[0, 1]],
                              sems[b]).wait()
        if with_deg:
            pltpu.make_async_copy(ones_v, deg_sh.at[sdbuf.at[0, 1]],
                                  semd[b]).wait()

    _start_gather(0, 0)

    @pl.loop(0, NCH_BASE // 2)
    def _pipe(j):
        i0 = 2 * j
        i1 = i0 + 1

        @pl.when(j > 0)
        def _():
            _wait_scatter(1)
        _start_gather(i1, 1)
        _wait_gather(0)
        _start_scatter(i0, 0)

        _wait_scatter(0)

        @pl.when(j < NCH_BASE // 2 - 1)
        def _():
            _start_gather(i0 + 2, 0)
        _wait_gather(1)
        _start_scatter(i1, 1)

    _wait_scatter(1)

    @pl.when(wid < NREM)
    def _():
        _start_gather(NCH_BASE, 0).wait()
        _start_scatter(NCH_BASE, 0)
        _wait_scatter(0)

    plsc.subcore_barrier()

    if with_deg:
        pltpu.sync_copy(deg_sh.at[pl.ds(rbase, ROWS_PER_SUB)], dv)

        def _expand(r, _):
            idx = jnp.full((16,), r, jnp.int32)
            dv16[r, :] = plsc.load_gather(dv, [idx])
            return 0
        lax.fori_loop(0, ROWS_PER_SUB, _expand, 0)

    acc_out = [acc0, acc1]
    for core in range(2):
        @pl.when(cid == core)
        def _():
            pltpu.sync_copy(acc_sh.at[pl.ds(rbase, ROWS_PER_SUB)],
                            acc_out[core].at[pl.ds(rbase, ROWS_PER_SUB)])
            if with_deg:
                deg_out = [deg0, deg1][core]
                pltpu.sync_copy(dv16,
                                deg_out.at[pl.ds(rbase, ROWS_PER_SUB)])


def _make_sc_pass(with_deg):
    out_type = [jax.ShapeDtypeStruct((NPAD, DH), jnp.float32),
                jax.ShapeDtypeStruct((NPAD, DH), jnp.float32)]
    if with_deg:
        out_type += [jax.ShapeDtypeStruct((NPAD, DH), jnp.float32),
                     jax.ShapeDtypeStruct((NPAD, DH), jnp.float32)]
    scratch = [pltpu.VMEM((NCH_BASE + 1, 2, CHUNK), jnp.int32),
               pltpu.VMEM((CHUNK, DH), jnp.float32),
               pltpu.VMEM((CHUNK, DH), jnp.float32)]
    if with_deg:
        scratch += [pltpu.VMEM((CHUNK,), jnp.float32)]
    scratch += [pltpu.VMEM((128, DH), jnp.float32)]
    if with_deg:
        scratch += [pltpu.VMEM((ROWS_PER_SUB,), jnp.float32),
                    pltpu.VMEM((ROWS_PER_SUB,), jnp.float32),
                    pltpu.VMEM((ROWS_PER_SUB, DH), jnp.float32)]
    scratch += [pltpu.VMEM_SHARED((NPAD, DH), jnp.float32)]
    if with_deg:
        scratch += [pltpu.VMEM_SHARED((NPAD,), jnp.float32)]
    nsem = 6 if with_deg else 4
    scratch += [pltpu.SemaphoreType.DMA] * nsem

    mesh = plsc.VectorSubcoreMesh(core_axis_name="c", subcore_axis_name="s")
    return pl.kernel(
        functools.partial(_sc_edge_pass_body, with_deg),
        out_type=out_type,
        mesh=mesh,
        scratch_types=scratch,
        compiler_params=pltpu.CompilerParams(use_tc_tiling_on_sc=False,
                                             needs_layout_passes=False),
        name=f"sc_edge_pass_deg{int(with_deg)}",
    )


_sc_pass_deg = _make_sc_pass(True)
_sc_pass = _make_sc_pass(False)


CROWS = NPAD // 8
NG = 8


def _place_cols(w, g, width):
    pieces = []
    if g > 0:
        pieces.append(jnp.zeros((w.shape[0], DH * g), jnp.float32))
    pieces.append(w)
    rest = width - DH * (g + 1)
    if rest > 0:
        pieces.append(jnp.zeros((w.shape[0], rest), jnp.float32))
    return jnp.concatenate(pieces, axis=1)


def _tc1_body(x_ref, w1l_ref, w1r_ref, y1_ref, z1_ref):
    x = x_ref[...]
    xp = jnp.concatenate(
        [x, jnp.zeros((NPAD - N, D_IN), jnp.float32)]).reshape(CROWS, NG, D_IN)
    wlr = jnp.concatenate([w1l_ref[...], w1r_ref[...]], axis=1)
    acc = jnp.zeros((CROWS, 256), jnp.float32)
    for g in range(NG):
        wg = jnp.concatenate(
            [_place_cols(wlr[:, :DH], g, 128), _place_cols(wlr[:, DH:], g, 128)],
            axis=1)
        acc = acc + jnp.dot(xp[:, g, :], wg,
                            preferred_element_type=jnp.float32)
    y1_ref[...] = acc[:, :128]
    z1_ref[...] = acc[:, 128:]


def _block_diag(w):
    return jnp.concatenate([_place_cols(w, g, 128) for g in range(NG)],
                           axis=0)


def _tc2_body(acc0_ref, acc1_ref, deg0_ref, deg1_ref, z1_ref, b1l_ref,
              w2l_ref, w2r_ref, y2_ref, z2_ref):
    acc = acc0_ref[...] + acc1_ref[...]
    deg = jnp.maximum(deg0_ref[...] + deg1_ref[...], 1.0)
    h = jnp.maximum(acc / deg + b1l_ref[...] + z1_ref[...], 0.0)
    bd2l = _block_diag(w2l_ref[...])
    bd2r = _block_diag(w2r_ref[...])
    y2_ref[...] = jnp.dot(h, bd2l, preferred_element_type=jnp.float32)
    z2_ref[...] = jnp.dot(h, bd2r, preferred_element_type=jnp.float32)


def _tc3_body(acc0_ref, acc1_ref, deg0_ref, deg1_ref, z2_ref, b2l_ref,
              out_ref):
    acc = acc0_ref[...] + acc1_ref[...]
    deg = jnp.maximum(deg0_ref[...] + deg1_ref[...], 1.0)
    out_ref[...] = acc / deg + b2l_ref[...] + z2_ref[...]


_tc1 = pl.pallas_call(
    _tc1_body,
    out_shape=[jax.ShapeDtypeStruct((CROWS, 128), jnp.float32),
               jax.ShapeDtypeStruct((CROWS, 128), jnp.float32)],
)

_tc2 = pl.pallas_call(
    _tc2_body,
    out_shape=[jax.ShapeDtypeStruct((CROWS, 128), jnp.float32),
               jax.ShapeDtypeStruct((CROWS, 128), jnp.float32)],
)

_tc3 = pl.pallas_call(
    _tc3_body,
    out_shape=jax.ShapeDtypeStruct((CROWS, 128), jnp.float32),
)


def kernel(x, edge_index, W1l, b1l, W1r, W2l, b2l, W2r):
    ei3 = edge_index.reshape(2, NCHUNK, CHUNK).transpose(1, 0, 2)

    W2l_p = jnp.pad(W2l, ((0, 0), (0, DH - W2l.shape[1])))
    W2r_p = jnp.pad(W2r, ((0, 0), (0, DH - W2r.shape[1])))
    b1l_p = jnp.tile(b1l, NG).reshape(1, 128)
    b2l_p = jnp.tile(jnp.pad(b2l, (0, DH - b2l.shape[0])), NG).reshape(1, 128)

    y1c, z1c = _tc1(x, W1l, W1r)
    a0, a1, d0, d1 = _sc_pass_deg(y1c.reshape(NPAD, DH), ei3)
    y2c, z2c = _tc2(a0.reshape(CROWS, 128), a1.reshape(CROWS, 128),
                    d0.reshape(CROWS, 128), d1.reshape(CROWS, 128),
                    z1c, b1l_p, W2l_p, W2r_p)
    b0, b1 = _sc_pass(y2c.reshape(NPAD, DH), ei3)
    out_c = _tc3(b0.reshape(CROWS, 128), b1.reshape(CROWS, 128),
                 d0.reshape(CROWS, 128), d1.reshape(CROWS, 128),
                 z2c, b2l_p)
    return out_c.reshape(NPAD, DH)[:N, :7]


# speedup vs baseline: 33.8283x; 1.2024x over previous
import functools

import jax
import jax.numpy as jnp
from jax import lax
from jax.experimental import pallas as pl
from jax.experimental.pallas import tpu as pltpu
from jax.experimental.pallas import tpu_sc as plsc

N = 10000
NPAD = 10240
E = 320000
D_IN = 128
DH = 16
CHUNK = 128
NCHUNK = E // CHUNK
NWORK = 32
NCH_BASE = NCHUNK // NWORK
NREM = NCHUNK - NCH_BASE * NWORK
ROWS_PER_SUB = NPAD // 16


def _sc_edge_pass_body(with_deg, table, ei3, *refs):
    if with_deg:
        (acc0, acc1, deg0, deg1, sdbuf, r0, r1, r2, r3, ones_v, zrow_v,
         zdeg_v, dv, dv16, acc_sh, deg_sh, g0, g1, g2, g3, s0, s1, s2, s3,
         e0, e1, e2, e3) = refs
        semd = [e0, e1, e2, e3]
    else:
        (acc0, acc1, sdbuf, r0, r1, r2, r3, zrow_v, acc_sh,
         g0, g1, g2, g3, s0, s1, s2, s3) = refs
        semd = None

    cid = lax.axis_index("c")
    sid = lax.axis_index("s")
    wid = sid * 2 + cid

    rows = [r0, r1, r2, r3]
    semg = [g0, g1, g2, g3]
    sems = [s0, s1, s2, s3]

    zf16 = jnp.zeros((16,), jnp.float32)

    def _zero_zrow(i, _):
        zrow_v[i, :] = zf16
        return 0
    lax.fori_loop(0, 128, _zero_zrow, 0)

    if with_deg:
        def _zero_zdeg(i, _):
            zdeg_v[pl.ds(i * 16, 16)] = zf16
            return 0
        lax.fori_loop(0, ROWS_PER_SUB // 16, _zero_zdeg, 0)

        of16 = jnp.ones((16,), jnp.float32)

        def _fill_ones(i, _):
            ones_v[pl.ds(i * 16, 16)] = of16
            return 0
        lax.fori_loop(0, CHUNK // 16, _fill_ones, 0)

    rbase = sid * ROWS_PER_SUB
    for j in range(ROWS_PER_SUB // 128):
        pltpu.sync_copy(zrow_v, acc_sh.at[pl.ds(rbase + j * 128, 128)])
    if with_deg:
        pltpu.sync_copy(zdeg_v, deg_sh.at[pl.ds(rbase, ROWS_PER_SUB)])

    cstart = wid * NCH_BASE
    pltpu.sync_copy(ei3.at[pl.ds(cstart, NCH_BASE)],
                    sdbuf.at[pl.ds(0, NCH_BASE)])

    @pl.when(wid < NREM)
    def _():
        extra = NWORK * NCH_BASE + wid
        pltpu.sync_copy(ei3.at[pl.ds(extra, 1)], sdbuf.at[pl.ds(NCH_BASE, 1)])

    plsc.subcore_barrier()

    def _start_gather(i, b):
        return pltpu.async_copy(table.at[sdbuf.at[i, 0]], rows[b], semg[b])

    def _wait_gather(b):
        pltpu.make_async_copy(table.at[sdbuf.at[0, 0]], rows[b],
                              semg[b]).wait()

    def _start_scatter(i, b):
        pltpu.async_copy(rows[b], acc_sh.at[sdbuf.at[i, 1]], sems[b], add=True)
        if with_deg:
            pltpu.async_copy(ones_v, deg_sh.at[sdbuf.at[i, 1]], semd[b],
                             add=True)

    def _wait_scatter(b):
        pltpu.make_async_copy(rows[b], acc_sh.at[sdbuf.at[0, 1]],
                              sems[b]).wait()
        if with_deg:
            pltpu.make_async_copy(ones_v, deg_sh.at[sdbuf.at[0, 1]],
                                  semd[b]).wait()

    _start_gather(0, 0)
    _start_gather(1, 1)

    @pl.loop(0, NCH_BASE // 4)
    def _pipe(j):
        for k in range(4):
            i = 4 * j + k

            @pl.when(i >= 2)
            def _():
                _wait_scatter((k + 2) % 4)

            @pl.when(i + 2 < NCH_BASE)
            def _():
                _start_gather(i + 2, (k + 2) % 4)
            _wait_gather(k)
            _start_scatter(i, k)

    for k in range(NCH_BASE % 4):
        i = NCH_BASE - (NCH_BASE % 4) + k
        _wait_scatter((k + 2) % 4)
        _wait_gather(k)
        _start_scatter(i, k)
    _wait_scatter((NCH_BASE - 2) % 4)
    _wait_scatter((NCH_BASE - 1) % 4)

    @pl.when(wid < NREM)
    def _():
        _start_gather(NCH_BASE, 0).wait()
        _start_scatter(NCH_BASE, 0)
        _wait_scatter(0)

    plsc.subcore_barrier()

    if with_deg:
        pltpu.sync_copy(deg_sh.at[pl.ds(rbase, ROWS_PER_SUB)], dv)

        def _expand(r, _):
            idx = jnp.full((16,), r, jnp.int32)
            dv16[r, :] = plsc.load_gather(dv, [idx])
            return 0
        lax.fori_loop(0, ROWS_PER_SUB, _expand, 0)

    acc_out = [acc0, acc1]
    for core in range(2):
        @pl.when(cid == core)
        def _():
            pltpu.sync_copy(acc_sh.at[pl.ds(rbase, ROWS_PER_SUB)],
                            acc_out[core].at[pl.ds(rbase, ROWS_PER_SUB)])
            if with_deg:
                deg_out = [deg0, deg1][core]
                pltpu.sync_copy(dv16,
                                deg_out.at[pl.ds(rbase, ROWS_PER_SUB)])


def _make_sc_pass(with_deg):
    out_type = [jax.ShapeDtypeStruct((NPAD, DH), jnp.float32),
                jax.ShapeDtypeStruct((NPAD, DH), jnp.float32)]
    if with_deg:
        out_type += [jax.ShapeDtypeStruct((NPAD, DH), jnp.float32),
                     jax.ShapeDtypeStruct((NPAD, DH), jnp.float32)]
    scratch = [pltpu.VMEM((NCH_BASE + 1, 2, CHUNK), jnp.int32)]
    scratch += [pltpu.VMEM((CHUNK, DH), jnp.float32)] * 4
    if with_deg:
        scratch += [pltpu.VMEM((CHUNK,), jnp.float32)]
    scratch += [pltpu.VMEM((128, DH), jnp.float32)]
    if with_deg:
        scratch += [pltpu.VMEM((ROWS_PER_SUB,), jnp.float32),
                    pltpu.VMEM((ROWS_PER_SUB,), jnp.float32),
                    pltpu.VMEM((ROWS_PER_SUB, DH), jnp.float32)]
    scratch += [pltpu.VMEM_SHARED((NPAD, DH), jnp.float32)]
    if with_deg:
        scratch += [pltpu.VMEM_SHARED((NPAD,), jnp.float32)]
    nsem = 12 if with_deg else 8
    scratch += [pltpu.SemaphoreType.DMA] * nsem

    mesh = plsc.VectorSubcoreMesh(core_axis_name="c", subcore_axis_name="s")
    return pl.kernel(
        functools.partial(_sc_edge_pass_body, with_deg),
        out_type=out_type,
        mesh=mesh,
        scratch_types=scratch,
        compiler_params=pltpu.CompilerParams(use_tc_tiling_on_sc=False,
                                             needs_layout_passes=False),
        name=f"sc_edge_pass_deg{int(with_deg)}",
    )


_sc_pass_deg = _make_sc_pass(True)
_sc_pass = _make_sc_pass(False)


CROWS = NPAD // 8
NG = 8


def _place_cols(w, g, width):
    pieces = []
    if g > 0:
        pieces.append(jnp.zeros((w.shape[0], DH * g), jnp.float32))
    pieces.append(w)
    rest = width - DH * (g + 1)
    if rest > 0:
        pieces.append(jnp.zeros((w.shape[0], rest), jnp.float32))
    return jnp.concatenate(pieces, axis=1)


def _tc1_body(x_ref, w1l_ref, w1r_ref, y1_ref, z1_ref):
    x = x_ref[...]
    xp = jnp.concatenate(
        [x, jnp.zeros((NPAD - N, D_IN), jnp.float32)]).reshape(CROWS, NG, D_IN)
    wlr = jnp.concatenate([w1l_ref[...], w1r_ref[...]], axis=1)
    acc = jnp.zeros((CROWS, 256), jnp.float32)
    for g in range(NG):
        wg = jnp.concatenate(
            [_place_cols(wlr[:, :DH], g, 128), _place_cols(wlr[:, DH:], g, 128)],
            axis=1)
        acc = acc + jnp.dot(xp[:, g, :], wg,
                            preferred_element_type=jnp.float32)
    y1_ref[...] = acc[:, :128]
    z1_ref[...] = acc[:, 128:]


def _block_diag(w):
    return jnp.concatenate([_place_cols(w, g, 128) for g in range(NG)],
                           axis=0)


def _tc2_body(acc0_ref, acc1_ref, deg0_ref, deg1_ref, z1_ref, b1l_ref,
              w2l_ref, w2r_ref, y2_ref, z2_ref):
    acc = acc0_ref[...] + acc1_ref[...]
    deg = jnp.maximum(deg0_ref[...] + deg1_ref[...], 1.0)
    h = jnp.maximum(acc / deg + b1l_ref[...] + z1_ref[...], 0.0)
    bd2l = _block_diag(w2l_ref[...])
    bd2r = _block_diag(w2r_ref[...])
    y2_ref[...] = jnp.dot(h, bd2l, preferred_element_type=jnp.float32)
    z2_ref[...] = jnp.dot(h, bd2r, preferred_element_type=jnp.float32)


def _tc3_body(acc0_ref, acc1_ref, deg0_ref, deg1_ref, z2_ref, b2l_ref,
              out_ref):
    acc = acc0_ref[...] + acc1_ref[...]
    deg = jnp.maximum(deg0_ref[...] + deg1_ref[...], 1.0)
    full = acc / deg + b2l_ref[...] + z2_ref[...]
    rowid = lax.broadcasted_iota(jnp.int32, (128, 64), 0)
    colid = lax.broadcasted_iota(jnp.int32, (128, 64), 1)
    sel = ((rowid % DH == colid % 8)
           & (rowid // DH == colid // 8)).astype(jnp.float32)
    packed = jnp.dot(full, sel, preferred_element_type=jnp.float32)
    parts = [packed[:, 8 * g:8 * (g + 1)][:, None, :] for g in range(NG)]
    out_ref[...] = jnp.concatenate(parts, axis=1).reshape(NPAD, 8)[:N, :7]


_tc1 = pl.pallas_call(
    _tc1_body,
    out_shape=[jax.ShapeDtypeStruct((CROWS, 128), jnp.float32),
               jax.ShapeDtypeStruct((CROWS, 128), jnp.float32)],
)

_tc2 = pl.pallas_call(
    _tc2_body,
    out_shape=[jax.ShapeDtypeStruct((CROWS, 128), jnp.float32),
               jax.ShapeDtypeStruct((CROWS, 128), jnp.float32)],
)

_tc3 = pl.pallas_call(
    _tc3_body,
    out_shape=jax.ShapeDtypeStruct((N, 7), jnp.float32),
)


def kernel(x, edge_index, W1l, b1l, W1r, W2l, b2l, W2r):
    ei3 = edge_index.reshape(2, NCHUNK, CHUNK).transpose(1, 0, 2)

    W2l_p = jnp.pad(W2l, ((0, 0), (0, DH - W2l.shape[1])))
    W2r_p = jnp.pad(W2r, ((0, 0), (0, DH - W2r.shape[1])))
    b1l_p = jnp.tile(b1l, NG).reshape(1, 128)
    b2l_p = jnp.tile(jnp.pad(b2l, (0, DH - b2l.shape[0])), NG).reshape(1, 128)

    y1c, z1c = _tc1(x, W1l, W1r)
    a0, a1, d0, d1 = _sc_pass_deg(y1c.reshape(NPAD, DH), ei3)
    y2c, z2c = _tc2(a0.reshape(CROWS, 128), a1.reshape(CROWS, 128),
                    d0.reshape(CROWS, 128), d1.reshape(CROWS, 128),
                    z1c, b1l_p, W2l_p, W2r_p)
    b0, b1 = _sc_pass(y2c.reshape(NPAD, DH), ei3)
    return _tc3(b0.reshape(CROWS, 128), b1.reshape(CROWS, 128),
                d0.reshape(CROWS, 128), d1.reshape(CROWS, 128),
                z2c, b2l_p)


# speedup vs baseline: 39.8230x; 1.1772x over previous
import functools

import jax
import jax.numpy as jnp
from jax import lax
from jax.experimental import pallas as pl
from jax.experimental.pallas import tpu as pltpu
from jax.experimental.pallas import tpu_sc as plsc

N = 10000
NPAD = 10240
E = 320000
D_IN = 128
DH = 16
CHUNK = 128
NCHUNK = E // CHUNK
NWORK = 32
NCH_BASE = NCHUNK // NWORK
NREM = NCHUNK - NCH_BASE * NWORK
ROWS_PER_SUB = NPAD // 16


def _sc_edge_pass_body(with_deg, table, ei3, *refs):
    if with_deg:
        (acc0, acc1, deg0, deg1, sdbuf, r0, r1, r2, r3, ones_v, zrow_v,
         zdeg_v, dv, dv16, acc_sh, deg_sh, tab_sh, g0, g1, g2, g3,
         s0, s1, s2, s3, e0, e1, e2, e3) = refs
        semd = [e0, e1, e2, e3]
    else:
        (acc0, acc1, sdbuf, r0, r1, r2, r3, zrow_v, acc_sh, tab_sh,
         g0, g1, g2, g3, s0, s1, s2, s3) = refs
        semd = None

    cid = lax.axis_index("c")
    sid = lax.axis_index("s")
    wid = sid * 2 + cid

    rows = [r0, r1, r2, r3]
    semg = [g0, g1, g2, g3]
    sems = [s0, s1, s2, s3]

    zf16 = jnp.zeros((16,), jnp.float32)

    def _zero_zrow(i, _):
        zrow_v[i, :] = zf16
        return 0
    lax.fori_loop(0, 128, _zero_zrow, 0)

    if with_deg:
        def _zero_zdeg(i, _):
            zdeg_v[pl.ds(i * 16, 16)] = zf16
            return 0
        lax.fori_loop(0, ROWS_PER_SUB // 16, _zero_zdeg, 0)

        of16 = jnp.ones((16,), jnp.float32)

        def _fill_ones(i, _):
            ones_v[pl.ds(i * 16, 16)] = of16
            return 0
        lax.fori_loop(0, CHUNK // 16, _fill_ones, 0)

    rbase = sid * ROWS_PER_SUB
    pltpu.sync_copy(table.at[pl.ds(rbase, ROWS_PER_SUB)],
                    tab_sh.at[pl.ds(rbase, ROWS_PER_SUB)])
    for j in range(ROWS_PER_SUB // 128):
        pltpu.sync_copy(zrow_v, acc_sh.at[pl.ds(rbase + j * 128, 128)])
    if with_deg:
        pltpu.sync_copy(zdeg_v, deg_sh.at[pl.ds(rbase, ROWS_PER_SUB)])

    cstart = wid * NCH_BASE
    pltpu.sync_copy(ei3.at[pl.ds(cstart, NCH_BASE)],
                    sdbuf.at[pl.ds(0, NCH_BASE)])

    @pl.when(wid < NREM)
    def _():
        extra = NWORK * NCH_BASE + wid
        pltpu.sync_copy(ei3.at[pl.ds(extra, 1)], sdbuf.at[pl.ds(NCH_BASE, 1)])

    plsc.subcore_barrier()

    def _start_gather(i, b):
        return pltpu.async_copy(tab_sh.at[sdbuf.at[i, 0]], rows[b], semg[b])

    def _wait_gather(b):
        pltpu.make_async_copy(tab_sh.at[sdbuf.at[0, 0]], rows[b],
                              semg[b]).wait()

    def _start_scatter(i, b):
        pltpu.async_copy(rows[b], acc_sh.at[sdbuf.at[i, 1]], sems[b], add=True)
        if with_deg:
            pltpu.async_copy(ones_v, deg_sh.at[sdbuf.at[i, 1]], semd[b],
                             add=True)

    def _wait_scatter(b):
        pltpu.make_async_copy(rows[b], acc_sh.at[sdbuf.at[0, 1]],
                              sems[b]).wait()
        if with_deg:
            pltpu.make_async_copy(ones_v, deg_sh.at[sdbuf.at[0, 1]],
                                  semd[b]).wait()

    _start_gather(0, 0)
    _start_gather(1, 1)

    @pl.loop(0, NCH_BASE // 4)
    def _pipe(j):
        for k in range(4):
            i = 4 * j + k

            @pl.when(i >= 2)
            def _():
                _wait_scatter((k + 2) % 4)

            @pl.when(i + 2 < NCH_BASE)
            def _():
                _start_gather(i + 2, (k + 2) % 4)
            _wait_gather(k)
            _start_scatter(i, k)

    for k in range(NCH_BASE % 4):
        i = NCH_BASE - (NCH_BASE % 4) + k
        _wait_scatter((k + 2) % 4)
        _wait_gather(k)
        _start_scatter(i, k)
    _wait_scatter((NCH_BASE - 2) % 4)
    _wait_scatter((NCH_BASE - 1) % 4)

    @pl.when(wid < NREM)
    def _():
        _start_gather(NCH_BASE, 0).wait()
        _start_scatter(NCH_BASE, 0)
        _wait_scatter(0)

    plsc.subcore_barrier()

    if with_deg:
        pltpu.sync_copy(deg_sh.at[pl.ds(rbase, ROWS_PER_SUB)], dv)

        def _expand(r, _):
            idx = jnp.full((16,), r, jnp.int32)
            dv16[r, :] = plsc.load_gather(dv, [idx])
            return 0
        lax.fori_loop(0, ROWS_PER_SUB, _expand, 0)

    acc_out = [acc0, acc1]
    for core in range(2):
        @pl.when(cid == core)
        def _():
            pltpu.sync_copy(acc_sh.at[pl.ds(rbase, ROWS_PER_SUB)],
                            acc_out[core].at[pl.ds(rbase, ROWS_PER_SUB)])
            if with_deg:
                deg_out = [deg0, deg1][core]
                pltpu.sync_copy(dv16,
                                deg_out.at[pl.ds(rbase, ROWS_PER_SUB)])


def _make_sc_pass(with_deg):
    out_type = [jax.ShapeDtypeStruct((NPAD, DH), jnp.float32),
                jax.ShapeDtypeStruct((NPAD, DH), jnp.float32)]
    if with_deg:
        out_type += [jax.ShapeDtypeStruct((NPAD, DH), jnp.float32),
                     jax.ShapeDtypeStruct((NPAD, DH), jnp.float32)]
    scratch = [pltpu.VMEM((NCH_BASE + 1, 2, CHUNK), jnp.int32)]
    scratch += [pltpu.VMEM((CHUNK, DH), jnp.float32)] * 4
    if with_deg:
        scratch += [pltpu.VMEM((CHUNK,), jnp.float32)]
    scratch += [pltpu.VMEM((128, DH), jnp.float32)]
    if with_deg:
        scratch += [pltpu.VMEM((ROWS_PER_SUB,), jnp.float32),
                    pltpu.VMEM((ROWS_PER_SUB,), jnp.float32),
                    pltpu.VMEM((ROWS_PER_SUB, DH), jnp.float32)]
    scratch += [pltpu.VMEM_SHARED((NPAD, DH), jnp.float32)]
    if with_deg:
        scratch += [pltpu.VMEM_SHARED((NPAD,), jnp.float32)]
    scratch += [pltpu.VMEM_SHARED((NPAD, DH), jnp.float32)]
    nsem = 12 if with_deg else 8
    scratch += [pltpu.SemaphoreType.DMA] * nsem

    mesh = plsc.VectorSubcoreMesh(core_axis_name="c", subcore_axis_name="s")
    return pl.kernel(
        functools.partial(_sc_edge_pass_body, with_deg),
        out_type=out_type,
        mesh=mesh,
        scratch_types=scratch,
        compiler_params=pltpu.CompilerParams(use_tc_tiling_on_sc=False,
                                             needs_layout_passes=False),
        name=f"sc_edge_pass_deg{int(with_deg)}",
    )


_sc_pass_deg = _make_sc_pass(True)
_sc_pass = _make_sc_pass(False)


CROWS = NPAD // 8
NG = 8


def _place_cols(w, g, width):
    pieces = []
    if g > 0:
        pieces.append(jnp.zeros((w.shape[0], DH * g), jnp.float32))
    pieces.append(w)
    rest = width - DH * (g + 1)
    if rest > 0:
        pieces.append(jnp.zeros((w.shape[0], rest), jnp.float32))
    return jnp.concatenate(pieces, axis=1)


def _tc1_body(x_ref, w1l_ref, w1r_ref, y1_ref, z1_ref):
    x = x_ref[...]
    xp = jnp.concatenate(
        [x, jnp.zeros((NPAD - N, D_IN), jnp.float32)]).reshape(CROWS, NG, D_IN)
    wlr = jnp.concatenate([w1l_ref[...], w1r_ref[...]], axis=1)
    acc = jnp.zeros((CROWS, 256), jnp.float32)
    for g in range(NG):
        wg = jnp.concatenate(
            [_place_cols(wlr[:, :DH], g, 128), _place_cols(wlr[:, DH:], g, 128)],
            axis=1)
        acc = acc + jnp.dot(xp[:, g, :], wg,
                            preferred_element_type=jnp.float32)
    y1_ref[...] = acc[:, :128]
    z1_ref[...] = acc[:, 128:]


def _block_diag(w):
    return jnp.concatenate([_place_cols(w, g, 128) for g in range(NG)],
                           axis=0)


def _tc2_body(acc0_ref, acc1_ref, deg0_ref, deg1_ref, z1_ref, b1l_ref,
              w2l_ref, w2r_ref, y2_ref, z2_ref):
    acc = acc0_ref[...] + acc1_ref[...]
    deg = jnp.maximum(deg0_ref[...] + deg1_ref[...], 1.0)
    h = jnp.maximum(acc / deg + b1l_ref[...] + z1_ref[...], 0.0)
    bd2l = _block_diag(w2l_ref[...])
    bd2r = _block_diag(w2r_ref[...])
    y2_ref[...] = jnp.dot(h, bd2l, preferred_element_type=jnp.float32)
    z2_ref[...] = jnp.dot(h, bd2r, preferred_element_type=jnp.float32)


def _tc3_body(acc0_ref, acc1_ref, deg0_ref, deg1_ref, z2_ref, b2l_ref,
              out_ref):
    acc = acc0_ref[...] + acc1_ref[...]
    deg = jnp.maximum(deg0_ref[...] + deg1_ref[...], 1.0)
    full = acc / deg + b2l_ref[...] + z2_ref[...]
    rowid = lax.broadcasted_iota(jnp.int32, (128, 64), 0)
    colid = lax.broadcasted_iota(jnp.int32, (128, 64), 1)
    sel = ((rowid % DH == colid % 8)
           & (rowid // DH == colid // 8)).astype(jnp.float32)
    packed = jnp.dot(full, sel, preferred_element_type=jnp.float32)
    parts = [packed[:, 8 * g:8 * (g + 1)][:, None, :] for g in range(NG)]
    out_ref[...] = jnp.concatenate(parts, axis=1).reshape(NPAD, 8)[:N, :7]


_tc1 = pl.pallas_call(
    _tc1_body,
    out_shape=[jax.ShapeDtypeStruct((CROWS, 128), jnp.float32),
               jax.ShapeDtypeStruct((CROWS, 128), jnp.float32)],
)

_tc2 = pl.pallas_call(
    _tc2_body,
    out_shape=[jax.ShapeDtypeStruct((CROWS, 128), jnp.float32),
               jax.ShapeDtypeStruct((CROWS, 128), jnp.float32)],
)

_tc3 = pl.pallas_call(
    _tc3_body,
    out_shape=jax.ShapeDtypeStruct((N, 7), jnp.float32),
)


def kernel(x, edge_index, W1l, b1l, W1r, W2l, b2l, W2r):
    ei3 = edge_index.reshape(2, NCHUNK, CHUNK).transpose(1, 0, 2)

    W2l_p = jnp.pad(W2l, ((0, 0), (0, DH - W2l.shape[1])))
    W2r_p = jnp.pad(W2r, ((0, 0), (0, DH - W2r.shape[1])))
    b1l_p = jnp.tile(b1l, NG).reshape(1, 128)
    b2l_p = jnp.tile(jnp.pad(b2l, (0, DH - b2l.shape[0])), NG).reshape(1, 128)

    y1c, z1c = _tc1(x, W1l, W1r)
    a0, a1, d0, d1 = _sc_pass_deg(y1c.reshape(NPAD, DH), ei3)
    y2c, z2c = _tc2(a0.reshape(CROWS, 128), a1.reshape(CROWS, 128),
                    d0.reshape(CROWS, 128), d1.reshape(CROWS, 128),
                    z1c, b1l_p, W2l_p, W2r_p)
    b0, b1 = _sc_pass(y2c.reshape(NPAD, DH), ei3)
    return _tc3(b0.reshape(CROWS, 128), b1.reshape(CROWS, 128),
                d0.reshape(CROWS, 128), d1.reshape(CROWS, 128),
                z2c, b2l_p)


# speedup vs baseline: 40.5907x; 1.0193x over previous
import functools

import jax
import jax.numpy as jnp
from jax import lax
from jax.experimental import pallas as pl
from jax.experimental.pallas import tpu as pltpu
from jax.experimental.pallas import tpu_sc as plsc

N = 10000
NPAD = 10240
E = 320000
D_IN = 128
DH = 16
CHUNK = 128
NCHUNK = E // CHUNK
NWORK = 32
NCH_BASE = NCHUNK // NWORK
NREM = NCHUNK - NCH_BASE * NWORK
ROWS_PER_SUB = NPAD // 16


def _sc_edge_pass_body(with_deg, table, ei3, *refs):
    if with_deg:
        (acc0, acc1, deg0, deg1, sdbuf, r0, r1, r2, r3, ones_v, zrow_v,
         zdeg_v, dv, dv16, acc_sh, deg_sh, tab_sh, g0, g1, g2, g3,
         s0, s1, s2, s3, e0, e1, e2, e3) = refs
        semd = [e0, e1, e2, e3]
    else:
        (acc0, acc1, sdbuf, r0, r1, r2, r3, zrow_v, acc_sh, tab_sh,
         g0, g1, g2, g3, s0, s1, s2, s3) = refs
        semd = None

    cid = lax.axis_index("c")
    sid = lax.axis_index("s")
    wid = sid * 2 + cid

    rows = [r0, r1, r2, r3]
    semg = [g0, g1, g2, g3]
    sems = [s0, s1, s2, s3]

    zf16 = jnp.zeros((16,), jnp.float32)

    def _zero_zrow(i, _):
        zrow_v[i, :] = zf16
        return 0
    lax.fori_loop(0, 128, _zero_zrow, 0)

    if with_deg:
        def _zero_zdeg(i, _):
            zdeg_v[pl.ds(i * 16, 16)] = zf16
            return 0
        lax.fori_loop(0, ROWS_PER_SUB // 16, _zero_zdeg, 0)

        of16 = jnp.ones((16,), jnp.float32)

        def _fill_ones(i, _):
            ones_v[pl.ds(i * 16, 16)] = of16
            return 0
        lax.fori_loop(0, CHUNK // 16, _fill_ones, 0)

    rbase = sid * ROWS_PER_SUB
    pltpu.sync_copy(table.at[pl.ds(rbase, ROWS_PER_SUB)],
                    tab_sh.at[pl.ds(rbase, ROWS_PER_SUB)])
    for j in range(ROWS_PER_SUB // 128):
        pltpu.sync_copy(zrow_v, acc_sh.at[pl.ds(rbase + j * 128, 128)])
    if with_deg:
        pltpu.sync_copy(zdeg_v, deg_sh.at[pl.ds(rbase, ROWS_PER_SUB)])

    cstart = wid * NCH_BASE
    pltpu.sync_copy(ei3.at[pl.ds(cstart, NCH_BASE)],
                    sdbuf.at[pl.ds(0, NCH_BASE)])

    @pl.when(wid < NREM)
    def _():
        extra = NWORK * NCH_BASE + wid
        pltpu.sync_copy(ei3.at[pl.ds(extra, 1)], sdbuf.at[pl.ds(NCH_BASE, 1)])

    plsc.subcore_barrier()

    def _start_gather(i, b):
        return pltpu.async_copy(tab_sh.at[sdbuf.at[i, 0]], rows[b], semg[b])

    def _wait_gather(b):
        pltpu.make_async_copy(tab_sh.at[sdbuf.at[0, 0]], rows[b],
                              semg[b]).wait()

    def _start_scatter(i, b):
        pltpu.async_copy(rows[b], acc_sh.at[sdbuf.at[i, 1]], sems[b], add=True)
        if with_deg:
            pltpu.async_copy(ones_v, deg_sh.at[sdbuf.at[i, 1]], semd[b],
                             add=True)

    def _wait_scatter(b):
        pltpu.make_async_copy(rows[b], acc_sh.at[sdbuf.at[0, 1]],
                              sems[b]).wait()
        if with_deg:
            pltpu.make_async_copy(ones_v, deg_sh.at[sdbuf.at[0, 1]],
                                  semd[b]).wait()

    _start_gather(0, 0)
    _start_gather(1, 1)

    @pl.loop(0, NCH_BASE // 4)
    def _pipe(j):
        for k in range(4):
            i = 4 * j + k

            @pl.when(i >= 2)
            def _():
                _wait_scatter((k + 2) % 4)

            @pl.when(i + 2 < NCH_BASE)
            def _():
                _start_gather(i + 2, (k + 2) % 4)
            _wait_gather(k)
            _start_scatter(i, k)

    for k in range(NCH_BASE % 4):
        i = NCH_BASE - (NCH_BASE % 4) + k
        _wait_scatter((k + 2) % 4)
        _wait_gather(k)
        _start_scatter(i, k)
    _wait_scatter((NCH_BASE - 2) % 4)
    _wait_scatter((NCH_BASE - 1) % 4)

    @pl.when(wid < NREM)
    def _():
        _start_gather(NCH_BASE, 0).wait()
        _start_scatter(NCH_BASE, 0)
        _wait_scatter(0)

    plsc.subcore_barrier()

    if with_deg:
        pltpu.sync_copy(deg_sh.at[pl.ds(rbase, ROWS_PER_SUB)], dv)

        def _expand(r, _):
            idx = jnp.full((16,), r, jnp.int32)
            dv16[r, :] = plsc.load_gather(dv, [idx])
            return 0
        lax.fori_loop(0, ROWS_PER_SUB, _expand, 0)

    acc_out = [acc0, acc1]
    for core in range(2):
        @pl.when(cid == core)
        def _():
            pltpu.sync_copy(acc_sh.at[pl.ds(rbase, ROWS_PER_SUB)],
                            acc_out[core].at[pl.ds(rbase, ROWS_PER_SUB)])
            if with_deg:
                deg_out = [deg0, deg1][core]
                pltpu.sync_copy(dv16,
                                deg_out.at[pl.ds(rbase, ROWS_PER_SUB)])


def _make_sc_pass(with_deg):
    out_type = [jax.ShapeDtypeStruct((NPAD, DH), jnp.float32),
                jax.ShapeDtypeStruct((NPAD, DH), jnp.float32)]
    if with_deg:
        out_type += [jax.ShapeDtypeStruct((NPAD, DH), jnp.float32),
                     jax.ShapeDtypeStruct((NPAD, DH), jnp.float32)]
    scratch = [pltpu.VMEM((NCH_BASE + 1, 2, CHUNK), jnp.int32)]
    scratch += [pltpu.VMEM((CHUNK, DH), jnp.float32)] * 4
    if with_deg:
        scratch += [pltpu.VMEM((CHUNK,), jnp.float32)]
    scratch += [pltpu.VMEM((128, DH), jnp.float32)]
    if with_deg:
        scratch += [pltpu.VMEM((ROWS_PER_SUB,), jnp.float32),
                    pltpu.VMEM((ROWS_PER_SUB,), jnp.float32),
                    pltpu.VMEM((ROWS_PER_SUB, DH), jnp.float32)]
    scratch += [pltpu.VMEM_SHARED((NPAD, DH), jnp.float32)]
    if with_deg:
        scratch += [pltpu.VMEM_SHARED((NPAD,), jnp.float32)]
    scratch += [pltpu.VMEM_SHARED((NPAD, DH), jnp.float32)]
    nsem = 12 if with_deg else 8
    scratch += [pltpu.SemaphoreType.DMA] * nsem

    mesh = plsc.VectorSubcoreMesh(core_axis_name="c", subcore_axis_name="s")
    return pl.kernel(
        functools.partial(_sc_edge_pass_body, with_deg),
        out_type=out_type,
        mesh=mesh,
        scratch_types=scratch,
        compiler_params=pltpu.CompilerParams(use_tc_tiling_on_sc=False,
                                             needs_layout_passes=False),
        name=f"sc_edge_pass_deg{int(with_deg)}",
    )


_sc_pass_deg = _make_sc_pass(True)
_sc_pass = _make_sc_pass(False)


CROWS = NPAD // 8
NG = 8


def _place_cols(w, g, width):
    pieces = []
    if g > 0:
        pieces.append(jnp.zeros((w.shape[0], DH * g), jnp.float32))
    pieces.append(w)
    rest = width - DH * (g + 1)
    if rest > 0:
        pieces.append(jnp.zeros((w.shape[0], rest), jnp.float32))
    return jnp.concatenate(pieces, axis=1)


def _tc1_body(x_ref, w1lr_ref, y1_ref, z1_ref):
    x = x_ref[...]
    xp = jnp.concatenate(
        [x, jnp.zeros((NPAD - N, D_IN), jnp.float32)]).reshape(CROWS, NG, D_IN)
    wlr = w1lr_ref[...]
    acc = jnp.zeros((CROWS, 256), jnp.float32)
    for g in range(NG):
        wg = jnp.concatenate(
            [_place_cols(wlr[:, :DH], g, 128), _place_cols(wlr[:, DH:], g, 128)],
            axis=1)
        acc = acc + jnp.dot(xp[:, g, :], wg,
                            preferred_element_type=jnp.float32)
    y1_ref[...] = acc[:, :128]
    z1_ref[...] = acc[:, 128:]


def _block_diag(w):
    return jnp.concatenate([_place_cols(w, g, 128) for g in range(NG)],
                           axis=0)


def _tc2_body(acc0_ref, acc1_ref, deg0_ref, deg1_ref, z1_ref, b1l_ref,
              w2lr_ref, y2_ref, z2_ref):
    acc = acc0_ref[...] + acc1_ref[...]
    deg = jnp.maximum(deg0_ref[...] + deg1_ref[...], 1.0)
    b1l = jnp.concatenate([b1l_ref[...]] * NG).reshape(1, 128)
    h = jnp.maximum(acc / deg + b1l + z1_ref[...], 0.0)
    w2lr = w2lr_ref[...]
    zpad = jnp.zeros((DH, DH - 7), jnp.float32)
    bd2l = _block_diag(jnp.concatenate([w2lr[:, :7], zpad], axis=1))
    bd2r = _block_diag(jnp.concatenate([w2lr[:, 7:], zpad], axis=1))
    y2_ref[...] = jnp.dot(h, bd2l, preferred_element_type=jnp.float32)
    z2_ref[...] = jnp.dot(h, bd2r, preferred_element_type=jnp.float32)


def _tc3_body(acc0_ref, acc1_ref, deg0_ref, deg1_ref, z2_ref, b2l_ref,
              out_ref):
    acc = acc0_ref[...] + acc1_ref[...]
    deg = jnp.maximum(deg0_ref[...] + deg1_ref[...], 1.0)
    b2l = jnp.concatenate(
        [jnp.concatenate([b2l_ref[...], jnp.zeros((DH - 7,), jnp.float32)])]
        * NG).reshape(1, 128)
    full = acc / deg + b2l + z2_ref[...]
    rowid = lax.broadcasted_iota(jnp.int32, (128, 64), 0)
    colid = lax.broadcasted_iota(jnp.int32, (128, 64), 1)
    sel = ((rowid % DH == colid % 8)
           & (rowid // DH == colid // 8)).astype(jnp.float32)
    packed = jnp.dot(full, sel, preferred_element_type=jnp.float32)
    parts = [packed[:, 8 * g:8 * (g + 1)][:, None, :] for g in range(NG)]
    out_ref[...] = jnp.concatenate(parts, axis=1).reshape(NPAD, 8)[:N, :7]


_tc1 = pl.pallas_call(
    _tc1_body,
    out_shape=[jax.ShapeDtypeStruct((CROWS, 128), jnp.float32),
               jax.ShapeDtypeStruct((CROWS, 128), jnp.float32)],
)

_tc2 = pl.pallas_call(
    _tc2_body,
    out_shape=[jax.ShapeDtypeStruct((CROWS, 128), jnp.float32),
               jax.ShapeDtypeStruct((CROWS, 128), jnp.float32)],
)

_tc3 = pl.pallas_call(
    _tc3_body,
    out_shape=jax.ShapeDtypeStruct((N, 7), jnp.float32),
)


def kernel(x, edge_index, W1l, b1l, W1r, W2l, b2l, W2r):
    ei3 = edge_index.reshape(2, NCHUNK, CHUNK).transpose(1, 0, 2)

    w1lr = jnp.concatenate([W1l, W1r], axis=1)
    w2lr = jnp.concatenate([W2l, W2r], axis=1)

    y1c, z1c = _tc1(x, w1lr)
    a0, a1, d0, d1 = _sc_pass_deg(y1c.reshape(NPAD, DH), ei3)
    y2c, z2c = _tc2(a0.reshape(CROWS, 128), a1.reshape(CROWS, 128),
                    d0.reshape(CROWS, 128), d1.reshape(CROWS, 128),
                    z1c, b1l, w2lr)
    b0, b1 = _sc_pass(y2c.reshape(NPAD, DH), ei3)
    return _tc3(b0.reshape(CROWS, 128), b1.reshape(CROWS, 128),
                d0.reshape(CROWS, 128), d1.reshape(CROWS, 128),
                z2c, b2l)


# speedup vs baseline: 42.4651x; 1.0462x over previous
import functools

import jax
import jax.numpy as jnp
from jax import lax
from jax.experimental import pallas as pl
from jax.experimental.pallas import tpu as pltpu
from jax.experimental.pallas import tpu_sc as plsc

N = 10000
NPAD = 10240
E = 320000
D_IN = 128
DH = 16
CHUNK = 128
NCHUNK = E // CHUNK
NWORK = 32
NCH_BASE = NCHUNK // NWORK
NREM = NCHUNK - NCH_BASE * NWORK
ROWS_PER_SUB = NPAD // 16


def _sc_edge_pass_body(with_deg, table, ei3, *refs):
    if with_deg:
        (acc0, acc1, deg0, deg1, sdbuf, r0, r1, r2, r3, ones_v, zrow_v,
         zdeg_v, dv, dv16, acc_sh, deg_sh, tab_sh, g0, g1, g2, g3,
         s0, s1, s2, s3, e0, e1, e2, e3) = refs
        semd = [e0, e1, e2, e3]
    else:
        (acc0, acc1, sdbuf, r0, r1, r2, r3, zrow_v, acc_sh, tab_sh,
         g0, g1, g2, g3, s0, s1, s2, s3) = refs
        semd = None

    cid = lax.axis_index("c")
    sid = lax.axis_index("s")
    wid = sid * 2 + cid

    rows = [r0, r1, r2, r3]
    semg = [g0, g1, g2, g3]
    sems = [s0, s1, s2, s3]

    zf16 = jnp.zeros((16,), jnp.float32)

    def _zero_zrow(i, _):
        zrow_v[i, :] = zf16
        return 0
    lax.fori_loop(0, 128, _zero_zrow, 0)

    if with_deg:
        def _zero_zdeg(i, _):
            zdeg_v[pl.ds(i * 16, 16)] = zf16
            return 0
        lax.fori_loop(0, ROWS_PER_SUB // 16, _zero_zdeg, 0)

        of16 = jnp.ones((16,), jnp.float32)

        def _fill_ones(i, _):
            ones_v[pl.ds(i * 16, 16)] = of16
            return 0
        lax.fori_loop(0, CHUNK // 16, _fill_ones, 0)

    rbase = sid * ROWS_PER_SUB
    cstart = wid * NCH_BASE
    pending = [
        pltpu.async_copy(table.at[pl.ds(rbase, ROWS_PER_SUB)],
                         tab_sh.at[pl.ds(rbase, ROWS_PER_SUB)], semg[0]),
        pltpu.async_copy(ei3.at[pl.ds(cstart, NCH_BASE)],
                         sdbuf.at[pl.ds(0, NCH_BASE)], semg[1]),
    ]
    for j in range(ROWS_PER_SUB // 128):
        pending.append(
            pltpu.async_copy(zrow_v, acc_sh.at[pl.ds(rbase + j * 128, 128)],
                             sems[j % 4]))
    if with_deg:
        pending.append(
            pltpu.async_copy(zdeg_v, deg_sh.at[pl.ds(rbase, ROWS_PER_SUB)],
                             semd[0]))

    @pl.when(wid < NREM)
    def _():
        extra = NWORK * NCH_BASE + wid
        pltpu.async_copy(ei3.at[pl.ds(extra, 1)], sdbuf.at[pl.ds(NCH_BASE, 1)],
                         semg[2]).wait()

    for p in pending:
        p.wait()

    plsc.subcore_barrier()

    def _start_gather(i, b):
        return pltpu.async_copy(tab_sh.at[sdbuf.at[i, 0]], rows[b], semg[b])

    def _wait_gather(b):
        pltpu.make_async_copy(tab_sh.at[sdbuf.at[0, 0]], rows[b],
                              semg[b]).wait()

    def _start_scatter(i, b):
        pltpu.async_copy(rows[b], acc_sh.at[sdbuf.at[i, 1]], sems[b], add=True)
        if with_deg:
            pltpu.async_copy(ones_v, deg_sh.at[sdbuf.at[i, 1]], semd[b],
                             add=True)

    def _wait_scatter(b):
        pltpu.make_async_copy(rows[b], acc_sh.at[sdbuf.at[0, 1]],
                              sems[b]).wait()
        if with_deg:
            pltpu.make_async_copy(ones_v, deg_sh.at[sdbuf.at[0, 1]],
                                  semd[b]).wait()

    _start_gather(0, 0)
    _start_gather(1, 1)

    @pl.loop(0, NCH_BASE // 4)
    def _pipe(j):
        for k in range(4):
            i = 4 * j + k

            @pl.when(i >= 2)
            def _():
                _wait_scatter((k + 2) % 4)

            @pl.when(i + 2 < NCH_BASE)
            def _():
                _start_gather(i + 2, (k + 2) % 4)
            _wait_gather(k)
            _start_scatter(i, k)

    for k in range(NCH_BASE % 4):
        i = NCH_BASE - (NCH_BASE % 4) + k
        _wait_scatter((k + 2) % 4)
        _wait_gather(k)
        _start_scatter(i, k)
    _wait_scatter((NCH_BASE - 2) % 4)
    _wait_scatter((NCH_BASE - 1) % 4)

    @pl.when(wid < NREM)
    def _():
        _start_gather(NCH_BASE, 0).wait()
        _start_scatter(NCH_BASE, 0)
        _wait_scatter(0)

    plsc.subcore_barrier()

    if with_deg:
        pltpu.sync_copy(deg_sh.at[pl.ds(rbase, ROWS_PER_SUB)], dv)

        def _expand(r, _):
            idx = jnp.full((16,), r, jnp.int32)
            dv16[r, :] = plsc.load_gather(dv, [idx])
            return 0
        lax.fori_loop(0, ROWS_PER_SUB, _expand, 0)

    acc_out = [acc0, acc1]
    for core in range(2):
        @pl.when(cid == core)
        def _():
            pltpu.sync_copy(acc_sh.at[pl.ds(rbase, ROWS_PER_SUB)],
                            acc_out[core].at[pl.ds(rbase, ROWS_PER_SUB)])
            if with_deg:
                deg_out = [deg0, deg1][core]
                pltpu.sync_copy(dv16,
                                deg_out.at[pl.ds(rbase, ROWS_PER_SUB)])


def _make_sc_pass(with_deg):
    out_type = [jax.ShapeDtypeStruct((NPAD, DH), jnp.float32),
                jax.ShapeDtypeStruct((NPAD, DH), jnp.float32)]
    if with_deg:
        out_type += [jax.ShapeDtypeStruct((NPAD, DH), jnp.float32),
                     jax.ShapeDtypeStruct((NPAD, DH), jnp.float32)]
    scratch = [pltpu.VMEM((NCH_BASE + 1, 2, CHUNK), jnp.int32)]
    scratch += [pltpu.VMEM((CHUNK, DH), jnp.float32)] * 4
    if with_deg:
        scratch += [pltpu.VMEM((CHUNK,), jnp.float32)]
    scratch += [pltpu.VMEM((128, DH), jnp.float32)]
    if with_deg:
        scratch += [pltpu.VMEM((ROWS_PER_SUB,), jnp.float32),
                    pltpu.VMEM((ROWS_PER_SUB,), jnp.float32),
                    pltpu.VMEM((ROWS_PER_SUB, DH), jnp.float32)]
    scratch += [pltpu.VMEM_SHARED((NPAD, DH), jnp.float32)]
    if with_deg:
        scratch += [pltpu.VMEM_SHARED((NPAD,), jnp.float32)]
    scratch += [pltpu.VMEM_SHARED((NPAD, DH), jnp.float32)]
    nsem = 12 if with_deg else 8
    scratch += [pltpu.SemaphoreType.DMA] * nsem

    mesh = plsc.VectorSubcoreMesh(core_axis_name="c", subcore_axis_name="s")
    return pl.kernel(
        functools.partial(_sc_edge_pass_body, with_deg),
        out_type=out_type,
        mesh=mesh,
        scratch_types=scratch,
        compiler_params=pltpu.CompilerParams(use_tc_tiling_on_sc=False,
                                             needs_layout_passes=False),
        name=f"sc_edge_pass_deg{int(with_deg)}",
    )


_sc_pass_deg = _make_sc_pass(True)
_sc_pass = _make_sc_pass(False)


CROWS = NPAD // 8
NG = 8


def _place_cols(w, g, width):
    pieces = []
    if g > 0:
        pieces.append(jnp.zeros((w.shape[0], DH * g), jnp.float32))
    pieces.append(w)
    rest = width - DH * (g + 1)
    if rest > 0:
        pieces.append(jnp.zeros((w.shape[0], rest), jnp.float32))
    return jnp.concatenate(pieces, axis=1)


def _tc1_body(x_ref, w1lr_ref, y1_ref, z1_ref):
    x = x_ref[...]
    xp = jnp.concatenate(
        [x, jnp.zeros((NPAD - N, D_IN), jnp.float32)]).reshape(CROWS, NG, D_IN)
    wlr = w1lr_ref[...]
    acc = jnp.zeros((CROWS, 256), jnp.float32)
    for g in range(NG):
        wg = jnp.concatenate(
            [_place_cols(wlr[:, :DH], g, 128), _place_cols(wlr[:, DH:], g, 128)],
            axis=1)
        acc = acc + jnp.dot(xp[:, g, :], wg,
                            preferred_element_type=jnp.float32)
    y1_ref[...] = acc[:, :128]
    z1_ref[...] = acc[:, 128:]


def _block_diag(w):
    return jnp.concatenate([_place_cols(w, g, 128) for g in range(NG)],
                           axis=0)


def _tc2_body(acc0_ref, acc1_ref, deg0_ref, deg1_ref, z1_ref, b1l_ref,
              w2lr_ref, y2_ref, z2_ref):
    acc = acc0_ref[...] + acc1_ref[...]
    deg = jnp.maximum(deg0_ref[...] + deg1_ref[...], 1.0)
    b1l = jnp.concatenate([b1l_ref[...]] * NG).reshape(1, 128)
    h = jnp.maximum(acc / deg + b1l + z1_ref[...], 0.0)
    w2lr = w2lr_ref[...]
    zpad = jnp.zeros((DH, DH - 7), jnp.float32)
    bd2l = _block_diag(jnp.concatenate([w2lr[:, :7], zpad], axis=1))
    bd2r = _block_diag(jnp.concatenate([w2lr[:, 7:], zpad], axis=1))
    y2_ref[...] = jnp.dot(h, bd2l, preferred_element_type=jnp.float32)
    z2_ref[...] = jnp.dot(h, bd2r, preferred_element_type=jnp.float32)


def _tc3_body(acc0_ref, acc1_ref, deg0_ref, deg1_ref, z2_ref, b2l_ref,
              out_ref):
    acc = acc0_ref[...] + acc1_ref[...]
    deg = jnp.maximum(deg0_ref[...] + deg1_ref[...], 1.0)
    b2l = jnp.concatenate(
        [jnp.concatenate([b2l_ref[...], jnp.zeros((DH - 7,), jnp.float32)])]
        * NG).reshape(1, 128)
    full = acc / deg + b2l + z2_ref[...]
    rowid = lax.broadcasted_iota(jnp.int32, (128, 64), 0)
    colid = lax.broadcasted_iota(jnp.int32, (128, 64), 1)
    sel = ((rowid % DH == colid % 8)
           & (rowid // DH == colid // 8)).astype(jnp.float32)
    packed = jnp.dot(full, sel, preferred_element_type=jnp.float32)
    parts = [packed[:, 8 * g:8 * (g + 1)][:, None, :] for g in range(NG)]
    out_ref[...] = jnp.concatenate(parts, axis=1).reshape(NPAD, 8)[:N, :7]


_tc1 = pl.pallas_call(
    _tc1_body,
    out_shape=[jax.ShapeDtypeStruct((CROWS, 128), jnp.float32),
               jax.ShapeDtypeStruct((CROWS, 128), jnp.float32)],
)

_tc2 = pl.pallas_call(
    _tc2_body,
    out_shape=[jax.ShapeDtypeStruct((CROWS, 128), jnp.float32),
               jax.ShapeDtypeStruct((CROWS, 128), jnp.float32)],
)

_tc3 = pl.pallas_call(
    _tc3_body,
    out_shape=jax.ShapeDtypeStruct((N, 7), jnp.float32),
)


def kernel(x, edge_index, W1l, b1l, W1r, W2l, b2l, W2r):
    ei3 = edge_index.reshape(2, NCHUNK, CHUNK).transpose(1, 0, 2)

    w1lr = jnp.concatenate([W1l, W1r], axis=1)
    w2lr = jnp.concatenate([W2l, W2r], axis=1)

    y1c, z1c = _tc1(x, w1lr)
    a0, a1, d0, d1 = _sc_pass_deg(y1c.reshape(NPAD, DH), ei3)
    y2c, z2c = _tc2(a0.reshape(CROWS, 128), a1.reshape(CROWS, 128),
                    d0.reshape(CROWS, 128), d1.reshape(CROWS, 128),
                    z1c, b1l, w2lr)
    b0, b1 = _sc_pass(y2c.reshape(NPAD, DH), ei3)
    return _tc3(b0.reshape(CROWS, 128), b1.reshape(CROWS, 128),
                d0.reshape(CROWS, 128), d1.reshape(CROWS, 128),
                z2c, b2l)
